# Initial kernel scaffold; baseline (speedup 1.0000x reference)
import jax
import jax.numpy as jnp
from jax.experimental import pallas as pl


def kernel(desc, word_embeds, weights):
    raise NotImplementedError("write your pallas kernel here")


# speedup vs baseline: 1.0270x; 1.0270x over previous
import jax
import jax.numpy as jnp
from jax import lax
from jax.experimental import pallas as pl
from jax.experimental.pallas import tpu as pltpu
from jax.experimental.pallas import tpu_sc as plsc

VOCAB = 1000000
EMBED_DIM = 64
BATCH = 4096
HIST = 50

NC, NS, L = 2, 16, 16
NW = NC * NS
RB = BATCH // NW
QT = RB * HIST
CHUNK = 2 * HIST
NCHUNK = QT // CHUNK
GROUP_ROWS = 8
GCHUNKS = GROUP_ROWS * HIST // CHUNK
NGROUPS = RB // GROUP_ROWS
GQ = GROUP_ROWS * HIST
NDV = EMBED_DIM // L
KFULL = HIST // L
TAIL = HIST - KFULL * L


def _body(desc_hbm, emb_hbm, w_hbm, out_hbm,
          desc_v, wbuf, probs, ebuf, out_v,
          wsem, esem0, esem1):
    wid = lax.axis_index("s") * NC + lax.axis_index("c")

    pltpu.sync_copy(desc_hbm.at[wid], desc_v)

    wcopies = [
        pltpu.async_copy(w_hbm.at[desc_v.at[c]], wbuf.at[c], wsem)
        for c in range(NCHUNK)
    ]

    def fire(g, bufbase, sem):
        for k in range(GCHUNKS):
            pltpu.async_copy(
                emb_hbm.at[desc_v.at[GCHUNKS * g + k]],
                ebuf.at[pl.ds(bufbase + k * CHUNK, CHUNK)],
                sem,
            )

    def wait_group(bufbase, sem):
        pltpu.make_async_copy(
            emb_hbm.at[pl.ds(0, GQ)],
            ebuf.at[pl.ds(bufbase, GQ)],
            sem,
        ).wait()

    fire(0, 0, esem0)
    fire(1, GQ, esem1)

    for c in wcopies:
        c.wait()

    def _warm(i, _):
        probs[0, pl.ds(0, L)] = jnp.full((L,), i, jnp.int32).astype(jnp.float32)
        return 0

    lax.fori_loop(0, 1, _warm, 0)

    tailmask = lax.iota(jnp.int32, L) >= ((KFULL + 1) * L - HIST)

    def softmax_row(r, _):
        chunk = lax.shift_right_logical(r, 1)
        off = jnp.bitwise_and(r, 1) * HIST
        w0 = wbuf[chunk, pl.ds(off, L)]
        w1 = wbuf[chunk, pl.ds(off + L, L)]
        w2 = wbuf[chunk, pl.ds(off + 2 * L, L)]
        w3 = wbuf[chunk, pl.ds(off + HIST - L, L)]
        m = jnp.max(jnp.maximum(jnp.maximum(w0, w1), jnp.maximum(w2, w3)))
        mb = jnp.full((L,), m, jnp.float32)
        e0 = jnp.exp(w0 - mb)
        e1 = jnp.exp(w1 - mb)
        e2 = jnp.exp(w2 - mb)
        e3 = jnp.exp(w3 - mb)
        s = (jnp.sum(e0 + e1 + e2)
             + jnp.sum(jnp.where(tailmask, e3, jnp.zeros((L,), jnp.float32))))
        ib = jnp.ones((L,), jnp.float32) / jnp.full((L,), s, jnp.float32)
        probs[r, pl.ds(0, L)] = e0 * ib
        probs[r, pl.ds(L, L)] = e1 * ib
        probs[r, pl.ds(2 * L, L)] = e2 * ib
        probs[r, pl.ds(HIST - L, L)] = e3 * ib
        return 0

    lax.fori_loop(0, RB, softmax_row, 0)

    def compute_group(g, bufbase):
        def row_step(r_loc, _):
            r = g * GROUP_ROWS + r_loc

            def accumulate(kbase, nt, acc):
                pe = probs[r, pl.ds(kbase, L)]
                qb = bufbase + r_loc * HIST + kbase
                for t in range(nt):
                    pb = jnp.full((L,), pe[t], jnp.float32)
                    for j in range(NDV):
                        acc[j] = acc[j] + pb * ebuf[qb + t, pl.ds(L * j, L)]
                return acc

            def k_step(k, _):
                acc = [out_v[r, pl.ds(L * j, L)] for j in range(NDV)]
                acc = accumulate(k * L, L, acc)
                for j in range(NDV):
                    out_v[r, pl.ds(L * j, L)] = acc[j]
                return 0

            for j in range(NDV):
                out_v[r, pl.ds(L * j, L)] = jnp.zeros((L,), jnp.float32)
            lax.fori_loop(0, KFULL, k_step, 0)

            acc = [out_v[r, pl.ds(L * j, L)] for j in range(NDV)]
            acc = accumulate(KFULL * L, TAIL, acc)
            for j in range(NDV):
                out_v[r, pl.ds(L * j, L)] = acc[j]
            return 0

        lax.fori_loop(0, GROUP_ROWS, row_step, 0)

    def pair_step(i, _):
        g0 = 2 * i
        wait_group(0, esem0)
        compute_group(g0, 0)
        fire(g0 + 2, 0, esem0)
        g1 = g0 + 1
        wait_group(GQ, esem1)
        compute_group(g1, GQ)
        fire(g1 + 2, GQ, esem1)
        return 0

    lax.fori_loop(0, NGROUPS // 2 - 1, pair_step, 0)

    wait_group(0, esem0)
    compute_group(NGROUPS - 2, 0)
    wait_group(GQ, esem1)
    compute_group(NGROUPS - 1, GQ)

    pltpu.sync_copy(out_v, out_hbm.at[pl.ds(wid * RB, RB)])


def kernel(desc, word_embeds, weights):
    desc_t = desc.reshape(NW, NCHUNK, CHUNK)
    w_flat = weights.reshape(VOCAB)
    mesh = plsc.VectorSubcoreMesh(
        core_axis_name="c", subcore_axis_name="s",
        num_cores=NC, num_subcores=NS,
    )
    f = pl.kernel(
        _body,
        out_type=jax.ShapeDtypeStruct((BATCH, EMBED_DIM), jnp.float32),
        mesh=mesh,
        compiler_params=pltpu.CompilerParams(
            use_tc_tiling_on_sc=False, needs_layout_passes=False),
        scratch_types=[
            pltpu.VMEM((NCHUNK, CHUNK), jnp.int32),
            pltpu.VMEM((NCHUNK, CHUNK), jnp.float32),
            pltpu.VMEM((RB, EMBED_DIM), jnp.float32),
            pltpu.VMEM((2 * GQ, EMBED_DIM), jnp.float32),
            pltpu.VMEM((RB, EMBED_DIM), jnp.float32),
            pltpu.SemaphoreType.DMA,
            pltpu.SemaphoreType.DMA,
            pltpu.SemaphoreType.DMA,
        ],
    )
    return f(desc_t, word_embeds, w_flat)


# speedup vs baseline: 1.0271x; 1.0001x over previous
import jax
import jax.numpy as jnp
from jax import lax
from jax.experimental import pallas as pl
from jax.experimental.pallas import tpu as pltpu
from jax.experimental.pallas import tpu_sc as plsc

VOCAB = 1000000
EMBED_DIM = 64
BATCH = 4096
HIST = 50

NC, NS, L = 2, 16, 16
NW = NC * NS
RB = BATCH // NW
QT = RB * HIST
CHUNK = 2 * HIST
NCHUNK = QT // CHUNK
GROUP_ROWS = 8
GCHUNKS = GROUP_ROWS * HIST // CHUNK
NGROUPS = RB // GROUP_ROWS
GQ = GROUP_ROWS * HIST
NDV = EMBED_DIM // L
KFULL = HIST // L
TAIL = HIST - KFULL * L


def _body(desc_hbm, emb_hbm, w_hbm, out_hbm,
          desc_v, wbuf, probs, ebuf, out_v,
          wsem, esem0, esem1):
    wid = lax.axis_index("s") * NC + lax.axis_index("c")

    pltpu.sync_copy(desc_hbm.at[wid], desc_v)

    wcopies = [
        pltpu.async_copy(w_hbm.at[desc_v.at[c]], wbuf.at[c], wsem)
        for c in range(NCHUNK)
    ]

    def fire(g, bufbase, sem):
        for k in range(GCHUNKS):
            pltpu.async_copy(
                emb_hbm.at[desc_v.at[GCHUNKS * g + k]],
                ebuf.at[pl.ds(bufbase + k * CHUNK, CHUNK)],
                sem,
            )

    def wait_group(bufbase, sem):
        pltpu.make_async_copy(
            emb_hbm.at[pl.ds(0, GQ)],
            ebuf.at[pl.ds(bufbase, GQ)],
            sem,
        ).wait()

    fire(0, 0, esem0)
    fire(1, GQ, esem1)

    for c in wcopies:
        c.wait()

    def _warm(i, _):
        probs[0, pl.ds(0, L)] = jnp.full((L,), i, jnp.int32).astype(jnp.float32)
        return 0

    lax.fori_loop(0, 1, _warm, 0)

    tailmask = lax.iota(jnp.int32, L) >= ((KFULL + 1) * L - HIST)

    def softmax_row(r, _):
        chunk = lax.shift_right_logical(r, 1)
        off = jnp.bitwise_and(r, 1) * HIST
        w0 = wbuf[chunk, pl.ds(off, L)]
        w1 = wbuf[chunk, pl.ds(off + L, L)]
        w2 = wbuf[chunk, pl.ds(off + 2 * L, L)]
        w3 = wbuf[chunk, pl.ds(off + HIST - L, L)]
        m = jnp.max(jnp.maximum(jnp.maximum(w0, w1), jnp.maximum(w2, w3)))
        mb = jnp.full((L,), m, jnp.float32)
        e0 = jnp.exp(w0 - mb)
        e1 = jnp.exp(w1 - mb)
        e2 = jnp.exp(w2 - mb)
        e3 = jnp.exp(w3 - mb)
        s = (jnp.sum(e0 + e1 + e2)
             + jnp.sum(jnp.where(tailmask, e3, jnp.zeros((L,), jnp.float32))))
        ib = jnp.ones((L,), jnp.float32) / jnp.full((L,), s, jnp.float32)
        probs[r, pl.ds(0, L)] = e0 * ib
        probs[r, pl.ds(L, L)] = e1 * ib
        probs[r, pl.ds(2 * L, L)] = e2 * ib
        probs[r, pl.ds(HIST - L, L)] = e3 * ib
        return 0

    lax.fori_loop(0, RB, softmax_row, 0)

    def compute_group(g, bufbase):
        def row_step(r_loc, _):
            r = g * GROUP_ROWS + r_loc

            def accumulate(kbase, nt, acc):
                pe = probs[r, pl.ds(kbase, L)]
                qb = bufbase + r_loc * HIST + kbase
                for t in range(nt):
                    pb = jnp.full((L,), pe[t], jnp.float32)
                    for j in range(NDV):
                        acc[j] = acc[j] + pb * ebuf[qb + t, pl.ds(L * j, L)]
                return acc

            def k_step(k, _):
                acc = [out_v[r, pl.ds(L * j, L)] for j in range(NDV)]
                acc = accumulate(k * L, L, acc)
                for j in range(NDV):
                    out_v[r, pl.ds(L * j, L)] = acc[j]
                return 0

            for j in range(NDV):
                out_v[r, pl.ds(L * j, L)] = jnp.zeros((L,), jnp.float32)
            lax.fori_loop(0, KFULL, k_step, 0)

            acc = [out_v[r, pl.ds(L * j, L)] for j in range(NDV)]
            acc = accumulate(KFULL * L, TAIL, acc)
            for j in range(NDV):
                out_v[r, pl.ds(L * j, L)] = acc[j]
            return 0

        lax.fori_loop(0, GROUP_ROWS, row_step, 0)

    def pair_step(i, _):
        g0 = 2 * i
        wait_group(0, esem0)
        compute_group(g0, 0)
        fire(g0 + 2, 0, esem0)
        g1 = g0 + 1
        wait_group(GQ, esem1)
        compute_group(g1, GQ)
        fire(g1 + 2, GQ, esem1)
        return 0

    lax.fori_loop(0, NGROUPS // 2 - 1, pair_step, 0)

    wait_group(0, esem0)
    compute_group(NGROUPS - 2, 0)
    wait_group(GQ, esem1)
    compute_group(NGROUPS - 1, GQ)

    pltpu.sync_copy(out_v, out_hbm.at[pl.ds(wid * RB, RB)])


def kernel(desc, word_embeds, weights):
    desc_t = desc.reshape(NW, NCHUNK, CHUNK)
    w_flat = weights[:, 0]
    mesh = plsc.VectorSubcoreMesh(
        core_axis_name="c", subcore_axis_name="s",
        num_cores=NC, num_subcores=NS,
    )
    f = pl.kernel(
        _body,
        out_type=jax.ShapeDtypeStruct((BATCH, EMBED_DIM), jnp.float32),
        mesh=mesh,
        compiler_params=pltpu.CompilerParams(
            use_tc_tiling_on_sc=False, needs_layout_passes=False),
        scratch_types=[
            pltpu.VMEM((NCHUNK, CHUNK), jnp.int32),
            pltpu.VMEM((NCHUNK, CHUNK), jnp.float32),
            pltpu.VMEM((RB, EMBED_DIM), jnp.float32),
            pltpu.VMEM((2 * GQ, EMBED_DIM), jnp.float32),
            pltpu.VMEM((RB, EMBED_DIM), jnp.float32),
            pltpu.SemaphoreType.DMA,
            pltpu.SemaphoreType.DMA,
            pltpu.SemaphoreType.DMA,
        ],
    )
    return f(desc_t, word_embeds, w_flat)


# speedup vs baseline: 1.0274x; 1.0002x over previous
import jax
import jax.numpy as jnp
from jax import lax
from jax.experimental import pallas as pl
from jax.experimental.pallas import tpu as pltpu
from jax.experimental.pallas import tpu_sc as plsc

VOCAB = 1000000
EMBED_DIM = 64
BATCH = 4096
HIST = 50

NC, NS, L = 2, 16, 16
NW = NC * NS
RB = BATCH // NW
GROUP_ROWS = 8
NGROUPS = RB // GROUP_ROWS
GQ = GROUP_ROWS * HIST
NDV = EMBED_DIM // L
KFULL = HIST // L
TAIL = HIST - KFULL * L


def _body(desc_hbm, emb_hbm, w_hbm, out_hbm,
          desc_v, wbuf, probs, ebuf, out_v,
          wsem, esem0, esem1):
    wid = lax.axis_index("s") * NC + lax.axis_index("c")

    pltpu.sync_copy(desc_hbm.at[pl.ds(wid * RB, RB)], desc_v)

    wcopies = [
        pltpu.async_copy(w_hbm.at[desc_v.at[r]], wbuf.at[r], wsem)
        for r in range(RB)
    ]

    def fire(g, bufbase, sem):
        for k in range(GROUP_ROWS):
            pltpu.async_copy(
                emb_hbm.at[desc_v.at[g * GROUP_ROWS + k]],
                ebuf.at[pl.ds(bufbase + k * HIST, HIST)],
                sem,
            )

    def wait_group(bufbase, sem):
        pltpu.make_async_copy(
            emb_hbm.at[pl.ds(0, GQ)],
            ebuf.at[pl.ds(bufbase, GQ)],
            sem,
        ).wait()

    fire(0, 0, esem0)
    fire(1, GQ, esem1)

    for c in wcopies:
        c.wait()

    tailmask = lax.iota(jnp.int32, L) >= ((KFULL + 1) * L - HIST)

    def softmax_row(r, _):
        w0 = wbuf[r, pl.ds(0, L)]
        w1 = wbuf[r, pl.ds(L, L)]
        w2 = wbuf[r, pl.ds(2 * L, L)]
        w3 = wbuf[r, pl.ds(HIST - L, L)]
        m = jnp.max(jnp.maximum(jnp.maximum(w0, w1), jnp.maximum(w2, w3)))
        mb = jnp.full((L,), m, jnp.float32)
        e0 = jnp.exp(w0 - mb)
        e1 = jnp.exp(w1 - mb)
        e2 = jnp.exp(w2 - mb)
        e3 = jnp.exp(w3 - mb)
        s = (jnp.sum(e0 + e1 + e2)
             + jnp.sum(jnp.where(tailmask, e3, jnp.zeros((L,), jnp.float32))))
        ib = jnp.ones((L,), jnp.float32) / jnp.full((L,), s, jnp.float32)
        probs[r, pl.ds(0, L)] = e0 * ib
        probs[r, pl.ds(L, L)] = e1 * ib
        probs[r, pl.ds(2 * L, L)] = e2 * ib
        probs[r, pl.ds(HIST - L, L)] = e3 * ib
        return 0

    lax.fori_loop(0, RB, softmax_row, 0)

    def compute_group(g, bufbase):
        def row_step(r_loc, _):
            r = g * GROUP_ROWS + r_loc

            def accumulate(kbase, nt, acc):
                pe = probs[r, pl.ds(kbase, L)]
                qb = bufbase + r_loc * HIST + kbase
                for t in range(nt):
                    pb = jnp.full((L,), pe[t], jnp.float32)
                    for j in range(NDV):
                        acc[j] = acc[j] + pb * ebuf[qb + t, pl.ds(L * j, L)]
                return acc

            def k_step(k, _):
                acc = [out_v[r, pl.ds(L * j, L)] for j in range(NDV)]
                acc = accumulate(k * L, L, acc)
                for j in range(NDV):
                    out_v[r, pl.ds(L * j, L)] = acc[j]
                return 0

            for j in range(NDV):
                out_v[r, pl.ds(L * j, L)] = jnp.zeros((L,), jnp.float32)
            lax.fori_loop(0, KFULL, k_step, 0)

            acc = [out_v[r, pl.ds(L * j, L)] for j in range(NDV)]
            acc = accumulate(KFULL * L, TAIL, acc)
            for j in range(NDV):
                out_v[r, pl.ds(L * j, L)] = acc[j]
            return 0

        lax.fori_loop(0, GROUP_ROWS, row_step, 0)

    def pair_step(i, _):
        g0 = 2 * i
        wait_group(0, esem0)
        compute_group(g0, 0)
        fire(g0 + 2, 0, esem0)
        g1 = g0 + 1
        wait_group(GQ, esem1)
        compute_group(g1, GQ)
        fire(g1 + 2, GQ, esem1)
        return 0

    lax.fori_loop(0, NGROUPS // 2 - 1, pair_step, 0)

    wait_group(0, esem0)
    compute_group(NGROUPS - 2, 0)
    wait_group(GQ, esem1)
    compute_group(NGROUPS - 1, GQ)

    pltpu.sync_copy(out_v, out_hbm.at[pl.ds(wid * RB, RB)])


def kernel(desc, word_embeds, weights):
    w_flat = weights[:, 0]
    mesh = plsc.VectorSubcoreMesh(
        core_axis_name="c", subcore_axis_name="s",
        num_cores=NC, num_subcores=NS,
    )
    f = pl.kernel(
        _body,
        out_type=jax.ShapeDtypeStruct((BATCH, EMBED_DIM), jnp.float32),
        mesh=mesh,
        compiler_params=pltpu.CompilerParams(
            use_tc_tiling_on_sc=False, needs_layout_passes=False),
        scratch_types=[
            pltpu.VMEM((RB, HIST), jnp.int32),
            pltpu.VMEM((RB, HIST), jnp.float32),
            pltpu.VMEM((RB, EMBED_DIM), jnp.float32),
            pltpu.VMEM((2 * GQ, EMBED_DIM), jnp.float32),
            pltpu.VMEM((RB, EMBED_DIM), jnp.float32),
            pltpu.SemaphoreType.DMA,
            pltpu.SemaphoreType.DMA,
            pltpu.SemaphoreType.DMA,
        ],
    )
    return f(desc, word_embeds, w_flat)
